# Initial kernel scaffold; baseline (speedup 1.0000x reference)
import jax
import jax.numpy as jnp
from jax.experimental import pallas as pl


def kernel(idx, targets, table):
    raise NotImplementedError("write your pallas kernel here")


# speedup vs baseline: 1.3835x; 1.3835x over previous
import functools

import jax
import jax.numpy as jnp
from jax import lax
from jax.experimental import pallas as pl
from jax.experimental.pallas import tpu as pltpu
from jax.experimental.pallas import tpu_sc as plsc

VOCAB = 1000
NTOK = 51200
NC, NS = 2, 16
NW = NC * NS
NPW = NTOK // NW
CHUNK = 80
NCHUNK = NPW // CHUNK


def _lse_body(table_ref, lse_ref):
    x = table_ref[...]
    m = jnp.max(x, axis=1, keepdims=True)
    s = jnp.sum(jnp.exp(x - m), axis=1, keepdims=True)
    lse_ref[...] = m + jnp.log(s)


def _row_lse(table):
    return pl.pallas_call(
        _lse_body,
        out_shape=jax.ShapeDtypeStruct((VOCAB, 1), jnp.float32),
    )(table)


_MESH = plsc.VectorSubcoreMesh(core_axis_name="c", subcore_axis_name="s")


@functools.partial(
    pl.kernel,
    mesh=_MESH,
    out_type=[
        jax.ShapeDtypeStruct((NTOK, VOCAB), jnp.float32),
        jax.ShapeDtypeStruct((NW, 16), jnp.float32),
    ],
    scratch_types=[
        pltpu.VMEM((CHUNK,), jnp.int32),
        pltpu.VMEM((CHUNK,), jnp.int32),
        pltpu.VMEM((CHUNK, VOCAB), jnp.float32),
        pltpu.VMEM((1024,), jnp.float32),
        pltpu.VMEM((16,), jnp.float32),
        pltpu.SemaphoreType.DMA,
    ],
    compiler_params=pltpu.CompilerParams(use_tc_tiling_on_sc=False,
                                         needs_layout_passes=False),
)
def _sc_gather(table_hbm, idx_hbm, tgt_hbm, lse_hbm,
               out_hbm, part_hbm,
               idx_v, tgt_v, rows_v, lse_v, acc_v, sem_r):
    wid = lax.axis_index("s") * NC + lax.axis_index("c")
    base = wid * NPW
    pltpu.sync_copy(lse_hbm, lse_v)

    def chunk_body(j, acc):
        off = base + j * CHUNK
        pltpu.sync_copy(idx_hbm.at[pl.ds(off, CHUNK)], idx_v)
        pltpu.sync_copy(tgt_hbm.at[pl.ds(off, CHUNK)], tgt_v)
        pltpu.async_copy(table_hbm.at[idx_v], rows_v, sem_r).wait()
        pltpu.sync_copy(rows_v, out_hbm.at[pl.ds(off, CHUNK)])
        for i in range(CHUNK // 16):
            s = pl.ds(i * 16, 16)
            rid = lax.iota(jnp.int32, 16) + (i * 16)
            vals = plsc.load_gather(rows_v, [rid, tgt_v[s]])
            lses = plsc.load_gather(lse_v, [idx_v[s]])
            acc = acc + (lses - vals)
        return acc

    acc = lax.fori_loop(0, NCHUNK, chunk_body, jnp.zeros((16,), jnp.float32))
    acc_v[...] = acc
    pltpu.sync_copy(acc_v, part_hbm.at[wid])


def _loss_body(part_ref, out_ref):
    out_ref[...] = (jnp.sum(part_ref[...]) / NTOK).reshape(1, 1)


def _loss_reduce(partials):
    return pl.pallas_call(
        _loss_body,
        out_shape=jax.ShapeDtypeStruct((1, 1), jnp.float32),
    )(partials)


def kernel(idx, targets, table):
    b, t = idx.shape
    flat_idx = idx.reshape(-1).astype(jnp.int32)
    flat_tgt = targets.reshape(-1).astype(jnp.int32)
    table = table.astype(jnp.float32)

    lse = _row_lse(table)
    lse_pad = jnp.pad(lse.reshape(VOCAB), (0, 1024 - VOCAB))

    out, part = _sc_gather(table, flat_idx, flat_tgt, lse_pad)
    loss = _loss_reduce(part)[0, 0]
    return (out.reshape(b, t, VOCAB), loss)


# speedup vs baseline: 1.4330x; 1.0357x over previous
import functools

import jax
import jax.numpy as jnp
from jax import lax
from jax.experimental import pallas as pl
from jax.experimental.pallas import tpu as pltpu
from jax.experimental.pallas import tpu_sc as plsc

VOCAB = 1000
NTOK = 51200
NC, NS = 2, 16
NW = NC * NS
NPW = NTOK // NW
CHUNK = 32
NCHUNK = NPW // CHUNK


def _lse_body(table_ref, lse_ref):
    x = table_ref[...]
    m = jnp.max(x, axis=1, keepdims=True)
    s = jnp.sum(jnp.exp(x - m), axis=1, keepdims=True)
    lse_ref[...] = m + jnp.log(s)


def _row_lse(table):
    return pl.pallas_call(
        _lse_body,
        out_shape=jax.ShapeDtypeStruct((VOCAB, 1), jnp.float32),
    )(table)


_MESH = plsc.VectorSubcoreMesh(core_axis_name="c", subcore_axis_name="s")


@functools.partial(
    pl.kernel,
    mesh=_MESH,
    out_type=[
        jax.ShapeDtypeStruct((NTOK, VOCAB), jnp.float32),
        jax.ShapeDtypeStruct((NW, 16), jnp.float32),
    ],
    scratch_types=[
        pltpu.VMEM((NPW,), jnp.int32),
        pltpu.VMEM((CHUNK,), jnp.int32),
        pltpu.VMEM((CHUNK,), jnp.int32),
        pltpu.VMEM((CHUNK, VOCAB), jnp.float32),
        pltpu.VMEM((CHUNK, VOCAB), jnp.float32),
        pltpu.VMEM((1024,), jnp.float32),
        pltpu.VMEM((16,), jnp.float32),
        pltpu.SemaphoreType.DMA,
        pltpu.SemaphoreType.DMA,
        pltpu.SemaphoreType.DMA,
        pltpu.SemaphoreType.DMA,
    ],
    compiler_params=pltpu.CompilerParams(use_tc_tiling_on_sc=False,
                                         needs_layout_passes=False),
)
def _sc_gather(table_hbm, packed_hbm, lse_hbm,
               out_hbm, part_hbm,
               packed_v, idx0, idx1, rows0, rows1, lse_v, acc_v,
               sg0, sg1, ss0, ss1):
    wid = lax.axis_index("s") * NC + lax.axis_index("c")
    base = wid * NPW
    pltpu.sync_copy(lse_hbm, lse_v)
    pltpu.sync_copy(packed_hbm.at[pl.ds(base, NPW)], packed_v)

    rows = (rows0, rows1)
    idxb = (idx0, idx1)
    sg = (sg0, sg1)
    ss = (ss0, ss1)

    def load_chunk_idx(j, b):
        for i in range(CHUNK // 16):
            src = pl.ds(j * CHUNK + i * 16, 16)
            dst = pl.ds(i * 16, 16)
            idxb[b][dst] = lax.shift_right_logical(packed_v[src], 10)

    def start_gather(b):
        return pltpu.async_copy(table_hbm.at[idxb[b]], rows[b], sg[b])

    def start_scatter(j, b):
        return pltpu.async_copy(
            rows[b], out_hbm.at[pl.ds(base + j * CHUNK, CHUNK)], ss[b])

    def loss_chunk(j, b, acc):
        for i in range(CHUNK // 16):
            sl = pl.ds(j * CHUNK + i * 16, 16)
            cid = lax.bitwise_and(packed_v[sl], 1023)
            rid = lax.iota(jnp.int32, 16) + i * 16
            vals = plsc.load_gather(rows[b], [rid, cid])
            lses = plsc.load_gather(lse_v, [idxb[b][pl.ds(i * 16, 16)]])
            acc = acc + (lses - vals)
        return acc

    gathers, scatters = {}, {}
    load_chunk_idx(0, 0)
    gathers[0] = start_gather(0)
    load_chunk_idx(1, 1)
    acc = jnp.zeros((16,), jnp.float32)
    for j in range(NCHUNK):
        cur = j & 1
        nxt = 1 - cur
        if j >= 1:
            scatters[j - 1].wait()
        if j + 1 < NCHUNK:
            gathers[j + 1] = start_gather(nxt)
        gathers[j].wait()
        scatters[j] = start_scatter(j, cur)
        acc = loss_chunk(j, cur, acc)
        if j + 2 < NCHUNK:
            load_chunk_idx(j + 2, cur)
    scatters[NCHUNK - 1].wait()

    acc_v[...] = acc
    pltpu.sync_copy(acc_v, part_hbm.at[wid])


def _loss_body(part_ref, out_ref):
    out_ref[...] = (jnp.sum(part_ref[...]) / NTOK).reshape(1, 1)


def _loss_reduce(partials):
    return pl.pallas_call(
        _loss_body,
        out_shape=jax.ShapeDtypeStruct((1, 1), jnp.float32),
    )(partials)


def kernel(idx, targets, table):
    b, t = idx.shape
    packed = (idx.reshape(-1).astype(jnp.int32) * 1024
              + targets.reshape(-1).astype(jnp.int32))
    table = table.astype(jnp.float32)

    lse = _row_lse(table)
    lse_pad = jnp.pad(lse.reshape(VOCAB), (0, 1024 - VOCAB))

    out, part = _sc_gather(table, packed, lse_pad)
    loss = _loss_reduce(part)[0, 0]
    return (out.reshape(b, t, VOCAB), loss)
